# baseline (speedup 1.0000x reference)
import functools

import jax
import jax.numpy as jnp
from jax import lax
from jax.experimental import pallas as pl
from jax.experimental.pallas import tpu as pltpu
from jax.experimental.pallas import tpu_sc as plsc

BATCH = 4096
SEQ = 200
DIM = 64
GDIM = 128
NUM_CORES = 2
NUM_SUBCORES = 16
NW = NUM_CORES * NUM_SUBCORES
BPW = BATCH // NW
XPAD = 1024
YSTRIDE = 208
XPW = XPAD // NW
TROWS = XPAD * YSTRIDE
LANE_ROW0 = 1000 * YSTRIDE
IDXW = SEQ + 8


def _build_body(x_pad, y_pad, l_tab, t_xy, xchunk, ybuf, ltv, bb0, bb1,
                lbuf, sem_b0, sem_b1):
    cid = lax.axis_index("c")
    sid = lax.axis_index("s")
    wid = sid * NUM_CORES + cid

    sem_b = [sem_b0, sem_b1]
    bb = [bb0, bb1]
    pltpu.sync_copy(x_pad.at[pl.ds(pl.multiple_of(wid * XPW, 8), XPW)],
                    xchunk)
    pltpu.sync_copy(y_pad, ybuf)

    def out_desc(k, slot):
        row0 = pl.multiple_of((wid * XPW + k) * YSTRIDE, 8)
        return pltpu.make_async_copy(bb[slot],
                                     t_xy.at[pl.ds(row0, YSTRIDE)],
                                     sem_b[slot])

    def k_body(k, carry):
        xv = [xchunk[k, pl.ds(q * 16, 16)] for q in range(4)]
        for slot in range(2):
            @pl.when(lax.rem(k, 2) == slot)
            def _():
                @pl.when(k >= 2)
                def _():
                    out_desc(k - 2, slot).wait()

                bslot = bb[slot]

                @plsc.parallel_loop(0, YSTRIDE, unroll=2)
                def _(yr):
                    for q in range(4):
                        sl = pl.ds(q * 16, 16)
                        bslot[yr, sl] = ybuf[yr, sl] + xv[q]

                out_desc(k, slot).start()
        return carry

    lax.fori_loop(0, XPW, k_body, 0)
    out_desc(XPW - 2, 0).wait()
    out_desc(XPW - 1, 1).wait()

    @pl.when(wid == NW - 1)
    def _():
        pltpu.sync_copy(l_tab, ltv)
        for r in range(8):
            for q in range(4):
                sl = pl.ds(q * 16, 16)
                lbuf[r, sl] = ltv[r, sl] * 10.0
        pltpu.sync_copy(lbuf, t_xy.at[pl.ds(LANE_ROW0, 8)])


_build_call = functools.partial(
    pl.kernel,
    mesh=plsc.VectorSubcoreMesh(core_axis_name="c", subcore_axis_name="s"),
    out_type=jax.ShapeDtypeStruct((TROWS, GDIM), jnp.float32),
    scratch_types=[
        pltpu.VMEM((XPW, DIM), jnp.float32),
        pltpu.VMEM((YSTRIDE, DIM), jnp.float32),
        pltpu.VMEM((8, DIM), jnp.float32),
        pltpu.VMEM((YSTRIDE, GDIM), jnp.float32),
        pltpu.VMEM((YSTRIDE, GDIM), jnp.float32),
        pltpu.VMEM((8, GDIM), jnp.float32),
        pltpu.SemaphoreType.DMA,
        pltpu.SemaphoreType.DMA,
    ],
)(_build_body)


def _main_body(idx_flat, t_xy, p_tab, out_hbm,
               it0, it1, it2, it3, pos_v, g0, g1, o0, o1,
               sem_x0, sem_x1,
               sem_o0, sem_o1,
               sem_t0, sem_t1, sem_t2, sem_t3):
    cid = lax.axis_index("c")
    sid = lax.axis_index("s")
    wid = sid * NUM_CORES + cid
    base_b = wid * BPW

    it = [it0, it1, it2, it3]
    gb = [g0, g1]
    ob = [o0, o1]
    sem_x = [sem_x0, sem_x1]
    sem_o = [sem_o0, sem_o1]
    sem_t = [sem_t0, sem_t1, sem_t2, sem_t3]

    pltpu.sync_copy(p_tab, pos_v)

    def tok_desc(j, t):
        off = pl.multiple_of((base_b + j) * IDXW, 8)
        return pltpu.make_async_copy(idx_flat.at[pl.ds(off, IDXW)],
                                     it[t], sem_t[t])

    def x_desc(t, p2):
        return pltpu.make_async_copy(t_xy.at[it[t]], gb[p2], sem_x[p2])

    def out_desc(j, p2):
        return pltpu.make_async_copy(ob[p2], out_hbm.at[base_b + j],
                                     sem_o[p2])

    for t in range(4):
        tok_desc(t, t).start()
    for t in range(2):
        tok_desc(t, t).wait()
        x_desc(t, t).start()

    def batch_body(j, carry):
        jm4 = lax.rem(j, 4)
        for ps in range(4):
            @pl.when(jm4 == ps)
            def _():
                p2 = ps % 2
                p4 = ps % 4
                gslot = gb[p2]
                oslot = ob[p2]
                x_desc(0, p2).wait()

                @pl.when(j >= 2)
                def _():
                    out_desc(j - 2, p2).wait()

                lane_vecs = [gslot[SEQ, pl.ds(q * 16, 16)]
                             for q in range(4)]

                @plsc.parallel_loop(0, SEQ, unroll=2)
                def _(r):
                    for q in range(4):
                        sl = pl.ds(q * 16, 16)
                        oslot[r, sl] = (gslot[r, sl] + pos_v[r, sl]
                                        + lane_vecs[q])

                out_desc(j, p2).start()

                @pl.when(j + 2 < BPW)
                def _():
                    tok_desc(j + 2, (p4 + 2) % 4).wait()
                    x_desc((p4 + 2) % 4, p2).start()

                @pl.when(j + 4 < BPW)
                def _():
                    tok_desc(j + 4, p4).start()
        return carry

    lax.fori_loop(0, BPW, batch_body, 0)

    out_desc(BPW - 2, (BPW - 2) % 2).wait()
    out_desc(BPW - 1, (BPW - 1) % 2).wait()


_main_call = functools.partial(
    pl.kernel,
    mesh=plsc.VectorSubcoreMesh(core_axis_name="c", subcore_axis_name="s"),
    out_type=jax.ShapeDtypeStruct((BATCH, SEQ, DIM), jnp.float32),
    scratch_types=[
        pltpu.VMEM((IDXW,), jnp.int32),
        pltpu.VMEM((IDXW,), jnp.int32),
        pltpu.VMEM((IDXW,), jnp.int32),
        pltpu.VMEM((IDXW,), jnp.int32),
        pltpu.VMEM((SEQ, DIM), jnp.float32),
        pltpu.VMEM((IDXW, GDIM), jnp.float32),
        pltpu.VMEM((IDXW, GDIM), jnp.float32),
        pltpu.VMEM((SEQ, DIM), jnp.float32),
        pltpu.VMEM((SEQ, DIM), jnp.float32),
        pltpu.SemaphoreType.DMA,
        pltpu.SemaphoreType.DMA,
        pltpu.SemaphoreType.DMA,
        pltpu.SemaphoreType.DMA,
        pltpu.SemaphoreType.DMA,
        pltpu.SemaphoreType.DMA,
        pltpu.SemaphoreType.DMA,
        pltpu.SemaphoreType.DMA,
    ],
)(_main_body)


@jax.jit
def kernel(x_tokens, y_tokens, lane_indices, x_table, y_table, pos_table,
           lane_table):
    x_tokens = x_tokens.astype(jnp.int32)
    y_tokens = y_tokens.astype(jnp.int32)
    lane_indices = lane_indices.astype(jnp.int32)
    idx2d = x_tokens * YSTRIDE + y_tokens
    lane_rows = jnp.broadcast_to((lane_indices + LANE_ROW0)[:, None],
                                 (BATCH, 8))
    idx_flat = jnp.concatenate([idx2d, lane_rows], axis=1).reshape(
        BATCH * IDXW)
    x_pad = jnp.pad(x_table, ((0, XPAD - x_table.shape[0]), (0, 0)))
    y_pad = jnp.pad(y_table, ((0, YSTRIDE - y_table.shape[0]), (0, 0)))
    t_xy = _build_call(x_pad, y_pad, lane_table)
    return _main_call(idx_flat, t_xy, pos_table)


# speedup vs baseline: 1.3827x; 1.3827x over previous
import functools

import jax
import jax.numpy as jnp
from jax import lax
from jax.experimental import pallas as pl
from jax.experimental.pallas import tpu as pltpu
from jax.experimental.pallas import tpu_sc as plsc

BATCH = 4096
SEQ = 200
DIM = 64
NUM_CORES = 2
NUM_SUBCORES = 16
NW = NUM_CORES * NUM_SUBCORES
BPW = BATCH // NW
XPAD = 1024
YSTRIDE = 208
XPW = XPAD // NW
TROWS = XPAD * YSTRIDE


def _build_body(x_pad, y_pad, t_xy, xchunk, ybuf, bbuf, sem_b0, sem_b1):
    cid = lax.axis_index("c")
    sid = lax.axis_index("s")
    wid = sid * NUM_CORES + cid

    sem_b = [sem_b0, sem_b1]
    pltpu.sync_copy(x_pad.at[pl.ds(wid * XPW, XPW)], xchunk)
    pltpu.sync_copy(y_pad, ybuf)

    def out_desc(k, slot):
        row0 = pl.multiple_of((wid * XPW + k) * YSTRIDE, 8)
        return pltpu.make_async_copy(bbuf.at[slot],
                                     t_xy.at[pl.ds(row0, YSTRIDE)],
                                     sem_b[slot])

    def k_body(k, carry):
        xv = [xchunk[k, pl.ds(q * 16, 16)] for q in range(4)]
        for slot in range(2):
            @pl.when(lax.rem(k, 2) == slot)
            def _():
                @pl.when(k >= 2)
                def _():
                    out_desc(k - 2, slot).wait()

                @plsc.parallel_loop(0, YSTRIDE, unroll=2)
                def _(yr):
                    for q in range(4):
                        sl = pl.ds(q * 16, 16)
                        bbuf[slot, yr, sl] = ybuf[yr, sl] + xv[q]

                out_desc(k, slot).start()
        return carry

    lax.fori_loop(0, XPW, k_body, 0)
    out_desc(XPW - 2, 0).wait()
    out_desc(XPW - 1, 1).wait()


_build_call = functools.partial(
    pl.kernel,
    mesh=plsc.VectorSubcoreMesh(core_axis_name="c", subcore_axis_name="s"),
    out_type=jax.ShapeDtypeStruct((TROWS, DIM), jnp.float32),
    scratch_types=[
        pltpu.VMEM((XPW, DIM), jnp.float32),
        pltpu.VMEM((YSTRIDE, DIM), jnp.float32),
        pltpu.VMEM((2, YSTRIDE, DIM), jnp.float32),
        pltpu.SemaphoreType.DMA,
        pltpu.SemaphoreType.DMA,
    ],
    compiler_params=pltpu.CompilerParams(use_tc_tiling_on_sc=False),
)(_build_body)


def _main_body(idx_flat, lane_idx_hbm, t_xy, p_tab, l_tab, out_hbm,
               it_ring, lane_idx, lane_rows, pos_v, obuf,
               sem_x0, sem_x1,
               sem_o0, sem_o1, sem_o2, sem_o3,
               sem_t0, sem_t1, sem_t2, sem_t3):
    cid = lax.axis_index("c")
    sid = lax.axis_index("s")
    wid = sid * NUM_CORES + cid
    base_b = wid * BPW

    sem_x = [sem_x0, sem_x1]
    sem_o = [sem_o0, sem_o1, sem_o2, sem_o3]
    sem_t = [sem_t0, sem_t1, sem_t2, sem_t3]

    pltpu.sync_copy(p_tab, pos_v)
    pltpu.sync_copy(lane_idx_hbm.at[pl.ds(base_b, BPW)], lane_idx)
    pltpu.async_copy(l_tab.at[lane_idx], lane_rows, sem_x0).wait()

    def tok_desc(j, t):
        off = pl.multiple_of((base_b + j) * SEQ, 8)
        return pltpu.make_async_copy(idx_flat.at[pl.ds(off, SEQ)],
                                     it_ring.at[t], sem_t[t])

    def x_desc(t, p4):
        return pltpu.make_async_copy(t_xy.at[it_ring.at[t]],
                                     obuf.at[p4], sem_x[p4 % 2])

    def out_desc(j, p4):
        return pltpu.make_async_copy(obuf.at[p4], out_hbm.at[base_b + j],
                                     sem_o[p4])

    for t in range(4):
        tok_desc(t, t).start()
    for t in range(2):
        tok_desc(t, t).wait()
        x_desc(t, t).start()

    def batch_body(j, carry):
        jm4 = lax.rem(j, 4)
        for ps in range(4):
            @pl.when(jm4 == ps)
            def _():
                x_desc(0, ps).wait()

                lane_vecs = [lane_rows[j, pl.ds(q * 16, 16)] * 10.0
                             for q in range(4)]

                @plsc.parallel_loop(0, SEQ, unroll=2)
                def _(r):
                    for q in range(4):
                        sl = pl.ds(q * 16, 16)
                        plsc.addupdate(obuf.at[ps, r, sl],
                                       pos_v[r, sl] + lane_vecs[q])

                out_desc(j, ps).start()

                ns = (ps + 2) % 4
                @pl.when(j + 2 < BPW)
                def _():
                    tok_desc(j + 2, ns).wait()

                    @pl.when(j >= 2)
                    def _():
                        out_desc(j - 2, ns).wait()

                    x_desc(ns, ns).start()

                @pl.when(j + 4 < BPW)
                def _():
                    tok_desc(j + 4, ps).start()
        return carry

    lax.fori_loop(0, BPW, batch_body, 0)

    for j in range(BPW - 4, BPW):
        out_desc(j, j % 4).wait()


_main_call = functools.partial(
    pl.kernel,
    mesh=plsc.VectorSubcoreMesh(core_axis_name="c", subcore_axis_name="s"),
    out_type=jax.ShapeDtypeStruct((BATCH, SEQ, DIM), jnp.float32),
    scratch_types=[
        pltpu.VMEM((4, SEQ), jnp.int32),
        pltpu.VMEM((BPW,), jnp.int32),
        pltpu.VMEM((BPW, DIM), jnp.float32),
        pltpu.VMEM((SEQ, DIM), jnp.float32),
        pltpu.VMEM((4, SEQ, DIM), jnp.float32),
        pltpu.SemaphoreType.DMA,
        pltpu.SemaphoreType.DMA,
        pltpu.SemaphoreType.DMA,
        pltpu.SemaphoreType.DMA,
        pltpu.SemaphoreType.DMA,
        pltpu.SemaphoreType.DMA,
        pltpu.SemaphoreType.DMA,
        pltpu.SemaphoreType.DMA,
        pltpu.SemaphoreType.DMA,
        pltpu.SemaphoreType.DMA,
    ],
    compiler_params=pltpu.CompilerParams(use_tc_tiling_on_sc=False),
)(_main_body)


@jax.jit
def kernel(x_tokens, y_tokens, lane_indices, x_table, y_table, pos_table,
           lane_table):
    x_tokens = x_tokens.astype(jnp.int32)
    y_tokens = y_tokens.astype(jnp.int32)
    lane_indices = lane_indices.astype(jnp.int32)
    idx_flat = (x_tokens * YSTRIDE + y_tokens).reshape(BATCH * SEQ)
    x_pad = jnp.pad(x_table, ((0, XPAD - x_table.shape[0]), (0, 0)))
    y_pad = jnp.pad(y_table, ((0, YSTRIDE - y_table.shape[0]), (0, 0)))
    t_xy = _build_call(x_pad, y_pad)
    return _main_call(idx_flat, lane_indices, t_xy, pos_table, lane_table)
